# Initial kernel scaffold; baseline (speedup 1.0000x reference)
import functools

import jax
import jax.numpy as jnp
from jax import lax
from jax.experimental import pallas as pl
from jax.experimental.pallas import tpu as pltpu
from jax.experimental.pallas import tpu_sc as plsc

N = 10000
E = 320000
D = 128
G = 64
BN_EPS = 1e-5

NC = 2
NS = 16
NW = NC * NS
CH = 128
NCHUNK = E // CH
ROWS_PER_TILE = N // NS


def _sc_msgpass_body(z_hbm, src_hbm, dst_hbm, out_hbm,
                     srcv, dstv, rows, agg_sh, gsem):
    c = lax.axis_index("c")
    s = lax.axis_index("s")
    w = c * NS + s

    def zbody(i, carry):
        for j in range(D // 16):
            rows[i, pl.ds(j * 16, 16)] = jnp.zeros((16,), jnp.float32)
        return carry
    lax.fori_loop(0, CH, zbody, 0)

    base = s * ROWS_PER_TILE
    nfull = ROWS_PER_TILE // CH
    rem = ROWS_PER_TILE - nfull * CH
    for k in range(nfull):
        pltpu.sync_copy(rows, agg_sh.at[pl.ds(base + k * CH, CH)])
    if rem:
        pltpu.sync_copy(rows.at[pl.ds(0, rem)],
                        agg_sh.at[pl.ds(base + nfull * CH, rem)])
    plsc.subcore_barrier()

    cnt = (NCHUNK - w + NW - 1) // NW

    def body(i, carry):
        off = (w + i * NW) * CH
        pltpu.sync_copy(src_hbm.at[pl.ds(off, CH)], srcv)
        pltpu.sync_copy(dst_hbm.at[pl.ds(off, CH)], dstv)
        pltpu.async_copy(z_hbm.at[srcv], rows, gsem).wait()
        pltpu.sync_copy(rows, agg_sh.at[dstv], add=True)
        return carry
    lax.fori_loop(0, cnt, body, 0)

    plsc.subcore_barrier()

    out_base = c * N + base
    pltpu.sync_copy(agg_sh.at[pl.ds(base, ROWS_PER_TILE)],
                    out_hbm.at[pl.ds(out_base, ROWS_PER_TILE)])


_sc_msgpass = functools.partial(
    pl.kernel,
    out_type=jax.ShapeDtypeStruct((NC * N, D), jnp.float32),
    mesh=plsc.VectorSubcoreMesh(core_axis_name="c", subcore_axis_name="s"),
    scratch_types=[
        pltpu.VMEM((CH,), jnp.int32),
        pltpu.VMEM((CH,), jnp.int32),
        pltpu.VMEM((CH, D), jnp.float32),
        pltpu.VMEM_SHARED((N, D), jnp.float32),
        pltpu.SemaphoreType.DMA,
    ],
)(_sc_msgpass_body)


def _mlp_body(z_ref, agg2_ref, w1_ref, b1_ref, w2_ref, b2_ref,
              g_ref, bt_ref, out_ref):
    h0 = z_ref[...] + agg2_ref[0:N] + agg2_ref[N:2 * N]
    r = jnp.dot(h0, w1_ref[...], preferred_element_type=jnp.float32,
                precision=lax.Precision.HIGHEST) + b1_ref[...]
    r = jnp.maximum(r, 0.0)
    h = jnp.dot(r, w2_ref[...], preferred_element_type=jnp.float32,
                precision=lax.Precision.HIGHEST) + b2_ref[...]
    mu = jnp.mean(h, axis=0, keepdims=True)
    dlt = h - mu
    var = jnp.mean(dlt * dlt, axis=0, keepdims=True)
    out_ref[...] = jnp.maximum(
        dlt * lax.rsqrt(var + BN_EPS) * g_ref[...] + bt_ref[...], 0.0)


_mlp_call = pl.pallas_call(
    _mlp_body,
    out_shape=jax.ShapeDtypeStruct((N, D), jnp.float32),
)


def _final_body(z_ref, agg2_ref, batch_ref, w1_ref, b1_ref, w2_ref, b2_ref,
                g_ref, bt_ref, out_ref, gout_ref):
    h0 = z_ref[...] + agg2_ref[0:N] + agg2_ref[N:2 * N]
    r = jnp.dot(h0, w1_ref[...], preferred_element_type=jnp.float32,
                precision=lax.Precision.HIGHEST) + b1_ref[...]
    r = jnp.maximum(r, 0.0)
    h = jnp.dot(r, w2_ref[...], preferred_element_type=jnp.float32,
                precision=lax.Precision.HIGHEST) + b2_ref[...]
    mu = jnp.mean(h, axis=0, keepdims=True)
    dlt = h - mu
    var = jnp.mean(dlt * dlt, axis=0, keepdims=True)
    zn = jnp.maximum(
        dlt * lax.rsqrt(var + BN_EPS) * g_ref[...] + bt_ref[...], 0.0)
    out_ref[...] = zn
    ids = lax.broadcasted_iota(jnp.int32, (G, N), 0)
    onehot = (ids == batch_ref[...]).astype(jnp.float32)
    seg = jnp.dot(onehot, zn, preferred_element_type=jnp.float32,
                  precision=lax.Precision.HIGHEST)
    cnts = jnp.sum(onehot, axis=1, keepdims=True)
    gout_ref[...] = seg / jnp.maximum(cnts, 1.0)


_final_call = pl.pallas_call(
    _final_body,
    out_shape=(jax.ShapeDtypeStruct((N, D), jnp.float32),
               jax.ShapeDtypeStruct((G, D), jnp.float32)),
)


def kernel(x, edge_index, batch, W1, b1, W2, b2, gamma, beta):
    src = edge_index[0]
    dst = edge_index[1]
    batch2d = batch.reshape(1, N)
    z = x
    for l in range(3):
        agg2 = _sc_msgpass(z, src, dst)
        w1 = W1[l]
        w2 = W2[l]
        b1l = b1[l].reshape(1, D)
        b2l = b2[l].reshape(1, D)
        gl = gamma[l].reshape(1, D)
        btl = beta[l].reshape(1, D)
        if l < 2:
            z = _mlp_call(z, agg2, w1, b1l, w2, b2l, gl, btl)
        else:
            z, g = _final_call(z, agg2, batch2d, w1, b1l, w2, b2l, gl, btl)
    return (z, g)


# speedup vs baseline: 5.7816x; 5.7816x over previous
import functools

import jax
import jax.numpy as jnp
from jax import lax
from jax.experimental import pallas as pl
from jax.experimental.pallas import tpu as pltpu
from jax.experimental.pallas import tpu_sc as plsc

N = 10000
E = 320000
D = 128
G = 64
BN_EPS = 1e-5

NC = 2
NS = 16
NW = NC * NS
CH = 128
NCHUNK = E // CH
ROWS_A = 632
ROWS_B = N - (NS - 1) * ROWS_A


def _sc_msgpass_body(z_hbm, src_hbm, dst_hbm, out_hbm,
                     srcv, dstv, rows, agg_sh, gsem):
    c = lax.axis_index("c")
    s = lax.axis_index("s")
    w = c * NS + s

    def zbody(i, carry):
        for j in range(D // 16):
            rows[i, pl.ds(j * 16, 16)] = jnp.zeros((16,), jnp.float32)
        return carry
    lax.fori_loop(0, CH, zbody, 0)

    base = s * ROWS_A

    def _zero_slice(nrows):
        k = 0
        while k + CH <= nrows:
            pltpu.sync_copy(rows, agg_sh.at[pl.ds(base + k, CH)])
            k += CH
        if nrows - k:
            pltpu.sync_copy(rows.at[pl.ds(0, nrows - k)],
                            agg_sh.at[pl.ds(base + k, nrows - k)])

    @pl.when(s < NS - 1)
    def _():
        _zero_slice(ROWS_A)

    @pl.when(s == NS - 1)
    def _():
        _zero_slice(ROWS_B)

    plsc.subcore_barrier()

    cnt = (NCHUNK - w + NW - 1) // NW

    def body(i, carry):
        off = (w + i * NW) * CH
        pltpu.sync_copy(src_hbm.at[pl.ds(off, CH)], srcv)
        pltpu.sync_copy(dst_hbm.at[pl.ds(off, CH)], dstv)
        pltpu.async_copy(z_hbm.at[srcv], rows, gsem).wait()
        pltpu.sync_copy(rows, agg_sh.at[dstv], add=True)
        return carry
    lax.fori_loop(0, cnt, body, 0)

    plsc.subcore_barrier()

    out_base = c * N + base

    @pl.when(s < NS - 1)
    def _():
        pltpu.sync_copy(agg_sh.at[pl.ds(base, ROWS_A)],
                        out_hbm.at[pl.ds(out_base, ROWS_A)])

    @pl.when(s == NS - 1)
    def _():
        pltpu.sync_copy(agg_sh.at[pl.ds(base, ROWS_B)],
                        out_hbm.at[pl.ds(out_base, ROWS_B)])


_sc_msgpass = functools.partial(
    pl.kernel,
    out_type=jax.ShapeDtypeStruct((NC * N, D), jnp.float32),
    mesh=plsc.VectorSubcoreMesh(core_axis_name="c", subcore_axis_name="s"),
    scratch_types=[
        pltpu.VMEM((CH,), jnp.int32),
        pltpu.VMEM((CH,), jnp.int32),
        pltpu.VMEM((CH, D), jnp.float32),
        pltpu.VMEM_SHARED((N, D), jnp.float32),
        pltpu.SemaphoreType.DMA,
    ],
)(_sc_msgpass_body)


def _mlp_body(z_ref, agg2_ref, w1_ref, b1_ref, w2_ref, b2_ref,
              g_ref, bt_ref, out_ref):
    h0 = z_ref[...] + agg2_ref[0:N] + agg2_ref[N:2 * N]
    w1b = w1_ref[...].astype(jnp.bfloat16)
    r = jnp.dot(h0.astype(jnp.bfloat16), w1b,
                preferred_element_type=jnp.float32) + b1_ref[...]
    r = jnp.maximum(r, 0.0)
    w2b = w2_ref[...].astype(jnp.bfloat16)
    h = jnp.dot(r.astype(jnp.bfloat16), w2b,
                preferred_element_type=jnp.float32) + b2_ref[...]
    mu = jnp.mean(h, axis=0, keepdims=True)
    dlt = h - mu
    var = jnp.mean(dlt * dlt, axis=0, keepdims=True)
    inv = lax.rsqrt(var + BN_EPS)
    inv = inv * (1.5 - 0.5 * (var + BN_EPS) * inv * inv)
    out_ref[...] = jnp.maximum(
        dlt * inv * g_ref[...] + bt_ref[...], 0.0)


_mlp_call = pl.pallas_call(
    _mlp_body,
    out_shape=jax.ShapeDtypeStruct((N, D), jnp.float32),
)


def _final_body(z_ref, agg2_ref, batch_ref, w1_ref, b1_ref, w2_ref, b2_ref,
                g_ref, bt_ref, out_ref, gout_ref):
    h0 = z_ref[...] + agg2_ref[0:N] + agg2_ref[N:2 * N]
    w1b = w1_ref[...].astype(jnp.bfloat16)
    r = jnp.dot(h0.astype(jnp.bfloat16), w1b,
                preferred_element_type=jnp.float32) + b1_ref[...]
    r = jnp.maximum(r, 0.0)
    w2b = w2_ref[...].astype(jnp.bfloat16)
    h = jnp.dot(r.astype(jnp.bfloat16), w2b,
                preferred_element_type=jnp.float32) + b2_ref[...]
    mu = jnp.mean(h, axis=0, keepdims=True)
    dlt = h - mu
    var = jnp.mean(dlt * dlt, axis=0, keepdims=True)
    inv = lax.rsqrt(var + BN_EPS)
    inv = inv * (1.5 - 0.5 * (var + BN_EPS) * inv * inv)
    zn = jnp.maximum(
        dlt * inv * g_ref[...] + bt_ref[...], 0.0)
    out_ref[...] = zn
    ids = lax.broadcasted_iota(jnp.int32, (G, N), 0)
    onehot = (ids == batch_ref[...]).astype(jnp.float32)
    seg = jnp.dot(onehot, zn, preferred_element_type=jnp.float32,
                  precision=lax.Precision.HIGHEST)
    cnts = jnp.sum(onehot, axis=1, keepdims=True)
    gout_ref[...] = seg / jnp.maximum(cnts, 1.0)


_final_call = pl.pallas_call(
    _final_body,
    out_shape=(jax.ShapeDtypeStruct((N, D), jnp.float32),
               jax.ShapeDtypeStruct((G, D), jnp.float32)),
)


def kernel(x, edge_index, batch, W1, b1, W2, b2, gamma, beta):
    src = edge_index[0]
    dst = edge_index[1]
    batch2d = batch.reshape(1, N)
    z = x
    for l in range(3):
        agg2 = _sc_msgpass(z, src, dst)
        w1 = W1[l]
        w2 = W2[l]
        b1l = b1[l].reshape(1, D)
        b2l = b2[l].reshape(1, D)
        gl = gamma[l].reshape(1, D)
        btl = beta[l].reshape(1, D)
        if l < 2:
            z = _mlp_call(z, agg2, w1, b1l, w2, b2l, gl, btl)
        else:
            z, g = _final_call(z, agg2, batch2d, w1, b1l, w2, b2l, gl, btl)
    return (z, g)


# speedup vs baseline: 10.1695x; 1.7589x over previous
import functools

import jax
import jax.numpy as jnp
from jax import lax
from jax.experimental import pallas as pl
from jax.experimental.pallas import tpu as pltpu
from jax.experimental.pallas import tpu_sc as plsc

N = 10000
E = 320000
D = 128
G = 64
BN_EPS = 1e-5

NC = 2
NS = 16
NW = NC * NS
CH = 128
NCHUNK = E // CH
NCPW = 80
PADC = NW * NCPW
NBUF = 4
ROWS_A = 632
ROWS_B = N - (NS - 1) * ROWS_A


def _sc_msgpass_body(z_hbm, src_hbm, dst_hbm, out_hbm,
                     src_all, dv0, dv1, dv2, dv3, rows, agg_sh, *sems):
    gsems = sems[0:2]
    ssems = sems[2:4]
    isems = sems[4:8]
    dv = [dv0, dv1, dv2, dv3]
    c = lax.axis_index("c")
    s = lax.axis_index("s")
    w = c * NS + s

    zbuf = rows.at[0]

    def zbody(i, carry):
        for j in range(D // 16):
            zbuf[i, pl.ds(j * 16, 16)] = jnp.zeros((16,), jnp.float32)
        return carry
    lax.fori_loop(0, CH, zbody, 0)

    base = s * ROWS_A

    def _zero_slice(nrows):
        k = 0
        while k + CH <= nrows:
            pltpu.sync_copy(zbuf, agg_sh.at[pl.ds(base + k, CH)])
            k += CH
        if nrows - k:
            pltpu.sync_copy(zbuf.at[pl.ds(0, nrows - k)],
                            agg_sh.at[pl.ds(base + k, nrows - k)])

    @pl.when(s < NS - 1)
    def _():
        _zero_slice(ROWS_A)

    @pl.when(s == NS - 1)
    def _():
        _zero_slice(ROWS_B)

    plsc.subcore_barrier()

    pltpu.sync_copy(src_hbm.at[pl.ds(w * NCPW, NCPW)], src_all)

    def fire_idx(k, j):
        pltpu.async_copy(dst_hbm.at[pl.ds((w * NCPW + k) * CH, CH)],
                         dv[j], isems[j])

    def wait_idx(j):
        pltpu.make_async_copy(dst_hbm.at[pl.ds(0, CH)], dv[j],
                              isems[j]).wait()

    def fire_gather(k, r):
        pltpu.async_copy(z_hbm.at[src_all.at[k]], rows.at[r], gsems[r])

    def wait_gather(r):
        pltpu.make_async_copy(z_hbm.at[pl.ds(0, CH)], rows.at[r],
                              gsems[r]).wait()

    def fire_scatter(r, j):
        pltpu.async_copy(rows.at[r], agg_sh.at[dv[j]], ssems[r], add=True)

    def wait_scatter(r):
        pltpu.make_async_copy(z_hbm.at[pl.ds(0, CH)], rows.at[0],
                              ssems[r]).wait()

    def _run(cnt):
        n4 = cnt // 4
        for j in range(3):
            fire_idx(j, j)
        fire_gather(0, 0)

        def body(i, carry):
            k0 = i * 4
            for j in range(4):
                k = k0 + j
                r = j % 2
                wait_gather(r)
                wait_idx(j)
                fire_scatter(r, j)
                if j == 0:
                    @pl.when(i > 0)
                    def _():
                        wait_scatter(1)
                else:
                    wait_scatter((j - 1) % 2)
                if j == 0:
                    fire_idx(k + 3, 3)
                else:
                    @pl.when(i < n4 - 1)
                    def _(k=k, j=j):
                        fire_idx(k + 3, (j + 3) % 4)
                if j < 3:
                    fire_gather(k + 1, (j + 1) % 2)
                else:
                    @pl.when(i < n4 - 1)
                    def _(k=k):
                        fire_gather(k + 1, 0)
            return carry
        lax.fori_loop(0, n4, body, 0)
        wait_scatter(1)

    @pl.when(w < NW - 1)
    def _():
        _run(NCPW)

    @pl.when(w == NW - 1)
    def _():
        _run(NCHUNK - (NW - 1) * NCPW)

    plsc.subcore_barrier()

    out_base = c * N + base

    @pl.when(s < NS - 1)
    def _():
        pltpu.sync_copy(agg_sh.at[pl.ds(base, ROWS_A)],
                        out_hbm.at[pl.ds(out_base, ROWS_A)])

    @pl.when(s == NS - 1)
    def _():
        pltpu.sync_copy(agg_sh.at[pl.ds(base, ROWS_B)],
                        out_hbm.at[pl.ds(out_base, ROWS_B)])


_sc_msgpass = functools.partial(
    pl.kernel,
    out_type=jax.ShapeDtypeStruct((NC * N, D), jnp.float32),
    mesh=plsc.VectorSubcoreMesh(core_axis_name="c", subcore_axis_name="s"),
    scratch_types=[
        pltpu.VMEM((NCPW, CH), jnp.int32),
        pltpu.VMEM((CH,), jnp.int32),
        pltpu.VMEM((CH,), jnp.int32),
        pltpu.VMEM((CH,), jnp.int32),
        pltpu.VMEM((CH,), jnp.int32),
        pltpu.VMEM((2, CH, D), jnp.float32),
        pltpu.VMEM_SHARED((N, D), jnp.float32),
    ] + [pltpu.SemaphoreType.DMA] * 8,
)(_sc_msgpass_body)


def _mlp_body(z_ref, agg2_ref, w1_ref, b1_ref, w2_ref, b2_ref,
              g_ref, bt_ref, out_ref):
    h0 = z_ref[...] + agg2_ref[0:N] + agg2_ref[N:2 * N]
    w1b = w1_ref[...].astype(jnp.bfloat16)
    r = jnp.dot(h0.astype(jnp.bfloat16), w1b,
                preferred_element_type=jnp.float32) + b1_ref[...]
    r = jnp.maximum(r, 0.0)
    w2b = w2_ref[...].astype(jnp.bfloat16)
    h = jnp.dot(r.astype(jnp.bfloat16), w2b,
                preferred_element_type=jnp.float32) + b2_ref[...]
    mu = jnp.mean(h, axis=0, keepdims=True)
    dlt = h - mu
    var = jnp.mean(dlt * dlt, axis=0, keepdims=True)
    inv = lax.rsqrt(var + BN_EPS)
    inv = inv * (1.5 - 0.5 * (var + BN_EPS) * inv * inv)
    out_ref[...] = jnp.maximum(
        dlt * inv * g_ref[...] + bt_ref[...], 0.0)


_mlp_call = pl.pallas_call(
    _mlp_body,
    out_shape=jax.ShapeDtypeStruct((N, D), jnp.float32),
)


def _final_body(z_ref, agg2_ref, batch_ref, w1_ref, b1_ref, w2_ref, b2_ref,
                g_ref, bt_ref, out_ref, gout_ref):
    h0 = z_ref[...] + agg2_ref[0:N] + agg2_ref[N:2 * N]
    w1b = w1_ref[...].astype(jnp.bfloat16)
    r = jnp.dot(h0.astype(jnp.bfloat16), w1b,
                preferred_element_type=jnp.float32) + b1_ref[...]
    r = jnp.maximum(r, 0.0)
    w2b = w2_ref[...].astype(jnp.bfloat16)
    h = jnp.dot(r.astype(jnp.bfloat16), w2b,
                preferred_element_type=jnp.float32) + b2_ref[...]
    mu = jnp.mean(h, axis=0, keepdims=True)
    dlt = h - mu
    var = jnp.mean(dlt * dlt, axis=0, keepdims=True)
    inv = lax.rsqrt(var + BN_EPS)
    inv = inv * (1.5 - 0.5 * (var + BN_EPS) * inv * inv)
    zn = jnp.maximum(
        dlt * inv * g_ref[...] + bt_ref[...], 0.0)
    out_ref[...] = zn
    ids = lax.broadcasted_iota(jnp.int32, (G, N), 0)
    onehot = (ids == batch_ref[...]).astype(jnp.float32)
    seg = jnp.dot(onehot, zn, preferred_element_type=jnp.float32,
                  precision=lax.Precision.HIGHEST)
    cnts = jnp.sum(onehot, axis=1, keepdims=True)
    gout_ref[...] = seg / jnp.maximum(cnts, 1.0)


_final_call = pl.pallas_call(
    _final_body,
    out_shape=(jax.ShapeDtypeStruct((N, D), jnp.float32),
               jax.ShapeDtypeStruct((G, D), jnp.float32)),
)


def kernel(x, edge_index, batch, W1, b1, W2, b2, gamma, beta):
    pad = PADC * CH - E
    src = jnp.pad(edge_index[0], (0, pad)).reshape(PADC, CH)
    dst = jnp.pad(edge_index[1], (0, pad))
    batch2d = batch.reshape(1, N)
    z = x
    for l in range(3):
        agg2 = _sc_msgpass(z, src, dst)
        w1 = W1[l]
        w2 = W2[l]
        b1l = b1[l].reshape(1, D)
        b2l = b2[l].reshape(1, D)
        gl = gamma[l].reshape(1, D)
        btl = beta[l].reshape(1, D)
        if l < 2:
            z = _mlp_call(z, agg2, w1, b1l, w2, b2l, gl, btl)
        else:
            z, g = _final_call(z, agg2, batch2d, w1, b1l, w2, b2l, gl, btl)
    return (z, g)
